# baseline (speedup 1.0000x reference)
import jax
import jax.numpy as jnp
from jax.experimental import pallas as pl
from jax.experimental.pallas import tpu as pltpu

IN_DIM = 1024
OUT_DIM = 1024
K = 2048
B_TOK = 8192
TM = 512


def _prep_body(a_ref, b_ref, a_bf_ref, b_bf_ref, inv_ref):
    a32 = a_ref[...]
    inv = jax.lax.rsqrt(jnp.sum(a32 * a32, axis=0, keepdims=True))
    inv_ref[...] = inv
    a_bf_ref[...] = a32.astype(jnp.bfloat16)
    b_bf_ref[...] = (b_ref[...] * inv.T).astype(jnp.bfloat16)


def _main_body(x_ref, a_bf_ref, b_bf_ref, inv_ref, out_ref, inner_ref):
    x_bf = x_ref[...].astype(jnp.bfloat16)
    inner_raw = jnp.dot(x_bf, a_bf_ref[...],
                        preferred_element_type=jnp.float32)
    inner_ref[...] = inner_raw * inv_ref[...]
    out_ref[...] = jnp.dot(inner_raw.astype(jnp.bfloat16), b_bf_ref[...],
                           preferred_element_type=jnp.float32)


def kernel(x, A, Bm):
    a_bf, b_bf, inv = pl.pallas_call(
        _prep_body,
        out_shape=[
            jax.ShapeDtypeStruct((IN_DIM, K), jnp.bfloat16),
            jax.ShapeDtypeStruct((K, OUT_DIM), jnp.bfloat16),
            jax.ShapeDtypeStruct((1, K), jnp.float32),
        ],
    )(A, Bm)

    n_tiles = B_TOK // TM
    out, inner = pl.pallas_call(
        _main_body,
        grid=(n_tiles,),
        in_specs=[
            pl.BlockSpec((TM, IN_DIM), lambda i: (i, 0)),
            pl.BlockSpec((IN_DIM, K), lambda i: (0, 0)),
            pl.BlockSpec((K, OUT_DIM), lambda i: (0, 0)),
            pl.BlockSpec((1, K), lambda i: (0, 0)),
        ],
        out_specs=[
            pl.BlockSpec((TM, OUT_DIM), lambda i: (i, 0)),
            pl.BlockSpec((TM, K), lambda i: (i, 0)),
        ],
        out_shape=[
            jax.ShapeDtypeStruct((B_TOK, OUT_DIM), jnp.float32),
            jax.ShapeDtypeStruct((B_TOK, K), jnp.float32),
        ],
    )(x, a_bf, b_bf, inv)
    return (out, inner)


# speedup vs baseline: 1.0569x; 1.0569x over previous
import jax
import jax.numpy as jnp
from jax.experimental import pallas as pl
from jax.experimental.pallas import tpu as pltpu

IN_DIM = 1024
OUT_DIM = 1024
K = 2048
B_TOK = 8192
TM = 512
CHUNKS = 2


def _fused_body(x_ref, a_ref, b_ref, out_ref, inner_ref,
                inv_norm_ref, a_bf_ref, b_bf_ref):
    step = pl.program_id(0)

    @pl.when(step == 0)
    def _prep():
        a32 = a_ref[...]
        inv = jax.lax.rsqrt(jnp.sum(a32 * a32, axis=0, keepdims=True))
        inv_norm_ref[...] = inv
        a_bf_ref[...] = a32.astype(jnp.bfloat16)
        b_bf_ref[...] = (b_ref[...] * inv.T).astype(jnp.bfloat16)

    inv = inv_norm_ref[...]
    h = TM // CHUNKS
    for c in range(CHUNKS):
        sl = pl.ds(c * h, h)
        x_bf = x_ref[sl, :].astype(jnp.bfloat16)
        raw = jnp.dot(x_bf, a_bf_ref[...],
                      preferred_element_type=jnp.float32)
        inner_ref[sl, :] = raw * inv
        out_ref[sl, :] = jnp.dot(raw.astype(jnp.bfloat16), b_bf_ref[...],
                                 preferred_element_type=jnp.float32)


def kernel(x, A, Bm):
    n_tiles = B_TOK // TM
    out, inner = pl.pallas_call(
        _fused_body,
        grid=(n_tiles,),
        in_specs=[
            pl.BlockSpec((TM, IN_DIM), lambda i: (i, 0)),
            pl.BlockSpec((IN_DIM, K), lambda i: (0, 0)),
            pl.BlockSpec((K, OUT_DIM), lambda i: (0, 0)),
        ],
        out_specs=[
            pl.BlockSpec((TM, OUT_DIM), lambda i: (i, 0)),
            pl.BlockSpec((TM, K), lambda i: (i, 0)),
        ],
        out_shape=[
            jax.ShapeDtypeStruct((B_TOK, OUT_DIM), jnp.float32),
            jax.ShapeDtypeStruct((B_TOK, K), jnp.float32),
        ],
        scratch_shapes=[
            pltpu.VMEM((1, K), jnp.float32),
            pltpu.VMEM((IN_DIM, K), jnp.bfloat16),
            pltpu.VMEM((K, OUT_DIM), jnp.bfloat16),
        ],
    )(x, A, Bm)
    return (out, inner)
